# Initial kernel scaffold; baseline (speedup 1.0000x reference)
import jax
import jax.numpy as jnp
from jax.experimental import pallas as pl
from jax.experimental.pallas import tpu as pltpu


def _silu(v):
    return v * jax.nn.sigmoid(v)


def _ln(v, w, b):
    m = v.mean(-1, keepdims=True)
    var = ((v - m) ** 2).mean(-1, keepdims=True)
    return (v - m) * jax.lax.rsqrt(var + 1e-5) * w + b


def _inproj_body(x_ref, w_ref, lnw_ref, lnb_ref, o_ref):
    xn = _ln(x_ref[...], lnw_ref[...], lnb_ref[...])
    o_ref[...] = jnp.dot(xn, w_ref[...], preferred_element_type=jnp.float32)


def _conv_body(xin_ref, convw_ref, convb_ref, xpw_ref, dtw_ref, dtb_ref,
               xs_ref, dt_ref, bp_ref, cp_ref, carry_ref):
    nb = pl.program_id(0)

    @pl.when(nb == 0)
    def _():
        carry_ref[...] = jnp.zeros_like(carry_ref)

    xin = xin_ref[...]
    ln = xin.shape[0]
    dc = convw_ref.shape[0]
    ext = jnp.concatenate([carry_ref[...], xin], axis=0)
    acc = jnp.broadcast_to(convb_ref[...], xin.shape)
    for k in range(dc):
        acc = acc + convw_ref[k, :][None, :] * ext[8 + k - (dc - 1): 8 + k - (dc - 1) + ln, :]
    xs = _silu(acc)
    xs_ref[...] = xs
    carry_ref[...] = xin[ln - 8: ln, :]
    xp = jnp.dot(xs, xpw_ref[...], preferred_element_type=jnp.float32)
    r = dtw_ref.shape[0]
    s = bp_ref.shape[1]
    bp_ref[...] = xp[:, r: r + s]
    cp_ref[...] = xp[:, r + s: r + 2 * s]
    dt_ref[...] = jax.nn.softplus(
        jnp.dot(xp[:, :r], dtw_ref[...], preferred_element_type=jnp.float32)
        + dtb_ref[...])


def _scan_body(xs_ref, dt_ref, bpt_ref, cpt_ref, alogt_ref, dp_ref, y_ref, h_ref):
    g = pl.program_id(0)

    @pl.when(g == 0)
    def _():
        h_ref[...] = jnp.zeros_like(h_ref)

    a_neg = -jnp.exp(alogt_ref[...])
    dp = dp_ref[...]
    lc = xs_ref.shape[0]

    def body(i, h):
        dt_t = dt_ref[pl.ds(i, 1), :]
        x_t = xs_ref[pl.ds(i, 1), :]
        b_t = bpt_ref[:, pl.ds(i, 1)]
        c_t = cpt_ref[:, pl.ds(i, 1)]
        da = jnp.exp(dt_t * a_neg)
        h = da * h + (dt_t * x_t) * b_t
        y_ref[pl.ds(i, 1), :] = jnp.sum(h * c_t, axis=0, keepdims=True) + dp * x_t
        return h

    h_ref[...] = jax.lax.fori_loop(0, lc, body, h_ref[...])


def _post_body(ys_ref, z_ref, opw_ref, x_ref, ln2w_ref, ln2b_ref, rw_ref,
               h_ref, hn_ref, comb_ref):
    y = ys_ref[...] * _silu(z_ref[...])
    h = x_ref[...] + jnp.dot(y, opw_ref[...], preferred_element_type=jnp.float32)
    h_ref[...] = h
    hn = _ln(h, ln2w_ref[...], ln2b_ref[...])
    hn_ref[...] = hn
    logits = jnp.dot(hn, rw_ref[...], preferred_element_type=jnp.float32)
    p = jax.nn.softmax(logits, axis=-1)
    col = jax.lax.broadcasted_iota(jnp.int32, p.shape, 1)
    i1 = jnp.argmax(p, axis=-1, keepdims=True)
    m1 = jnp.max(p, axis=-1, keepdims=True)
    pm = jnp.where(col == i1, -jnp.inf, p)
    i2 = jnp.argmax(pm, axis=-1, keepdims=True)
    m2 = jnp.max(pm, axis=-1, keepdims=True)
    tot = m1 + m2
    comb_ref[...] = jnp.where(col == i1, m1 / tot,
                              jnp.where(col == i2, m2 / tot, 0.0))


def _moe_body(hn_ref, wg_ref, wu_ref, wd_ref, comb_ref, h_ref, o_ref):
    e = pl.program_id(1)
    hb = pl.program_id(2)

    @pl.when((e == 0) & (hb == 0))
    def _():
        o_ref[...] = h_ref[...]

    xb = hn_ref[...]
    dn = (((1,), (1,)), ((), ()))
    g = jax.lax.dot_general(xb, wg_ref[0], dn, preferred_element_type=jnp.float32)
    u = jax.lax.dot_general(xb, wu_ref[0], dn, preferred_element_type=jnp.float32)
    act = _silu(g) * u
    eo = jax.lax.dot_general(act, wd_ref[0], dn, preferred_element_type=jnp.float32)
    w_col = jax.lax.dynamic_slice(comb_ref[...], (0, e), (xb.shape[0], 1))
    o_ref[...] += eo * w_col


def kernel(x, ln1_w, ln1_b, ln2_w, ln2_b, in_proj_w, conv_w, conv_b, x_proj_w,
           dt_proj_w, dt_proj_b, A_log, Dp, out_proj_w, router_w, w_gate, w_up, w_down):
    b, t, d = x.shape
    di = conv_w.shape[0]
    r = dt_proj_w.shape[1]
    s = A_log.shape[1]
    e = router_w.shape[0]
    h_dim = w_gate.shape[1]
    n = b * t
    f32 = jnp.float32
    xf = x.reshape(n, d)

    ln = 256
    cb = 768
    n_nb = n // ln

    xz = pl.pallas_call(
        _inproj_body,
        grid=(n_nb, (2 * di) // cb),
        in_specs=[
            pl.BlockSpec((ln, d), lambda i, j: (i, 0)),
            pl.BlockSpec((d, cb), lambda i, j: (0, j)),
            pl.BlockSpec((1, d), lambda i, j: (0, 0)),
            pl.BlockSpec((1, d), lambda i, j: (0, 0)),
        ],
        out_specs=pl.BlockSpec((ln, cb), lambda i, j: (i, j)),
        out_shape=jax.ShapeDtypeStruct((n, 2 * di), f32),
    )(xf, in_proj_w.T, ln1_w.reshape(1, d), ln1_b.reshape(1, d))

    xs, dt, bp, cp = pl.pallas_call(
        _conv_body,
        grid=(n_nb,),
        in_specs=[
            pl.BlockSpec((ln, di), lambda i: (i, 0)),
            pl.BlockSpec((conv_w.shape[1], di), lambda i: (0, 0)),
            pl.BlockSpec((1, di), lambda i: (0, 0)),
            pl.BlockSpec((di, r + 2 * s), lambda i: (0, 0)),
            pl.BlockSpec((r, di), lambda i: (0, 0)),
            pl.BlockSpec((1, di), lambda i: (0, 0)),
        ],
        out_specs=[
            pl.BlockSpec((ln, di), lambda i: (i, 0)),
            pl.BlockSpec((ln, di), lambda i: (i, 0)),
            pl.BlockSpec((ln, s), lambda i: (i, 0)),
            pl.BlockSpec((ln, s), lambda i: (i, 0)),
        ],
        out_shape=[
            jax.ShapeDtypeStruct((n, di), f32),
            jax.ShapeDtypeStruct((n, di), f32),
            jax.ShapeDtypeStruct((n, s), f32),
            jax.ShapeDtypeStruct((n, s), f32),
        ],
        scratch_shapes=[pltpu.VMEM((8, di), f32)],
    )(xz, conv_w.T, conv_b.reshape(1, di), x_proj_w.T, dt_proj_w.T,
      dt_proj_b.reshape(1, di))

    lc = 256
    ys = pl.pallas_call(
        _scan_body,
        grid=(n // lc,),
        in_specs=[
            pl.BlockSpec((lc, di), lambda i: (i, 0)),
            pl.BlockSpec((lc, di), lambda i: (i, 0)),
            pl.BlockSpec((s, lc), lambda i: (0, i)),
            pl.BlockSpec((s, lc), lambda i: (0, i)),
            pl.BlockSpec((s, di), lambda i: (0, 0)),
            pl.BlockSpec((1, di), lambda i: (0, 0)),
        ],
        out_specs=pl.BlockSpec((lc, di), lambda i: (i, 0)),
        out_shape=jax.ShapeDtypeStruct((n, di), f32),
        scratch_shapes=[pltpu.VMEM((s, di), f32)],
    )(xs, dt, bp.T, cp.T, A_log.T, Dp.reshape(1, di))

    h, hn, comb = pl.pallas_call(
        _post_body,
        grid=(n_nb,),
        in_specs=[
            pl.BlockSpec((ln, di), lambda i: (i, 0)),
            pl.BlockSpec((ln, di), lambda i: (i, 1)),
            pl.BlockSpec((di, d), lambda i: (0, 0)),
            pl.BlockSpec((ln, d), lambda i: (i, 0)),
            pl.BlockSpec((1, d), lambda i: (0, 0)),
            pl.BlockSpec((1, d), lambda i: (0, 0)),
            pl.BlockSpec((d, e), lambda i: (0, 0)),
        ],
        out_specs=[
            pl.BlockSpec((ln, d), lambda i: (i, 0)),
            pl.BlockSpec((ln, d), lambda i: (i, 0)),
            pl.BlockSpec((ln, e), lambda i: (i, 0)),
        ],
        out_shape=[
            jax.ShapeDtypeStruct((n, d), f32),
            jax.ShapeDtypeStruct((n, d), f32),
            jax.ShapeDtypeStruct((n, e), f32),
        ],
    )(ys, xz, out_proj_w.T, xf, ln2_w.reshape(1, d), ln2_b.reshape(1, d),
      router_w.T)

    bn = 512
    bh = 512
    out = pl.pallas_call(
        _moe_body,
        grid=(n // bn, e, h_dim // bh),
        in_specs=[
            pl.BlockSpec((bn, d), lambda i, j, k: (i, 0)),
            pl.BlockSpec((1, bh, d), lambda i, j, k: (j, k, 0)),
            pl.BlockSpec((1, bh, d), lambda i, j, k: (j, k, 0)),
            pl.BlockSpec((1, d, bh), lambda i, j, k: (j, 0, k)),
            pl.BlockSpec((bn, e), lambda i, j, k: (i, 0)),
            pl.BlockSpec((bn, d), lambda i, j, k: (i, 0)),
        ],
        out_specs=pl.BlockSpec((bn, d), lambda i, j, k: (i, 0)),
        out_shape=jax.ShapeDtypeStruct((n, d), f32),
    )(hn, w_gate, w_up, w_down, comb, h)

    return out.reshape(b, t, d)


# speedup vs baseline: 8.9193x; 8.9193x over previous
import jax
import jax.numpy as jnp
from jax.experimental import pallas as pl
from jax.experimental.pallas import tpu as pltpu


def _silu(v):
    return v * jax.nn.sigmoid(v)


def _ln(v, w, b):
    m = v.mean(-1, keepdims=True)
    var = ((v - m) ** 2).mean(-1, keepdims=True)
    return (v - m) * jax.lax.rsqrt(var + 1e-5) * w + b


def _inproj_body(x_ref, w_ref, lnw_ref, lnb_ref, o_ref):
    xn = _ln(x_ref[...], lnw_ref[...], lnb_ref[...])
    o_ref[...] = jnp.dot(xn, w_ref[...], preferred_element_type=jnp.float32)


def _conv_body(xin_ref, convw_ref, convb_ref, xpw_ref, dtw_ref, dtb_ref,
               xs_ref, dt_ref, bp_ref, cp_ref, carry_ref):
    nb = pl.program_id(0)

    @pl.when(nb == 0)
    def _():
        carry_ref[...] = jnp.zeros_like(carry_ref)

    xin = xin_ref[...]
    ln = xin.shape[0]
    dc = convw_ref.shape[0]
    ext = jnp.concatenate([carry_ref[...], xin], axis=0)
    acc = jnp.broadcast_to(convb_ref[...], xin.shape)
    for k in range(dc):
        acc = acc + convw_ref[k, :][None, :] * ext[8 + k - (dc - 1): 8 + k - (dc - 1) + ln, :]
    xs = _silu(acc)
    xs_ref[...] = xs
    carry_ref[...] = xin[ln - 8: ln, :]
    xp = jnp.dot(xs, xpw_ref[...], preferred_element_type=jnp.float32)
    r = dtw_ref.shape[0]
    s = bp_ref.shape[1]
    bp_ref[...] = xp[:, r: r + s]
    cp_ref[...] = xp[:, r + s: r + 2 * s]
    dt_ref[...] = jax.nn.softplus(
        jnp.dot(xp[:, :r], dtw_ref[...], preferred_element_type=jnp.float32)
        + dtb_ref[...])


def _scan_body(xs_ref, dt_ref, bp_ref, cp_ref, alogt_ref, dp_ref, y_ref, h_ref):
    g = pl.program_id(0)

    @pl.when(g == 0)
    def _():
        h_ref[...] = jnp.zeros_like(h_ref)

    a_neg = -jnp.exp(alogt_ref[...])
    dp = dp_ref[...]
    lc = xs_ref.shape[0]
    outer_dn = (((0,), (0,)), ((), ()))
    contr_dn = (((1,), (0,)), ((), ()))

    def body(i, h):
        dt_t = dt_ref[pl.ds(i, 1), :]
        x_t = xs_ref[pl.ds(i, 1), :]
        b_t = bp_ref[pl.ds(i, 1), :]
        c_t = cp_ref[pl.ds(i, 1), :]
        da = jnp.exp(dt_t * a_neg)
        dbx = jax.lax.dot_general(b_t, dt_t * x_t, outer_dn,
                                  preferred_element_type=jnp.float32)
        h = da * h + dbx
        y_t = jax.lax.dot_general(c_t, h, contr_dn,
                                  preferred_element_type=jnp.float32)
        y_ref[pl.ds(i, 1), :] = y_t + dp * x_t
        return h

    h_ref[...] = jax.lax.fori_loop(0, lc, body, h_ref[...])


def _post_body(ys_ref, z_ref, opw_ref, x_ref, ln2w_ref, ln2b_ref, rw_ref,
               h_ref, hn_ref, comb_ref):
    y = ys_ref[...] * _silu(z_ref[...])
    h = x_ref[...] + jnp.dot(y, opw_ref[...], preferred_element_type=jnp.float32)
    h_ref[...] = h
    hn = _ln(h, ln2w_ref[...], ln2b_ref[...])
    hn_ref[...] = hn
    logits = jnp.dot(hn, rw_ref[...], preferred_element_type=jnp.float32)
    p = jax.nn.softmax(logits, axis=-1)
    col = jax.lax.broadcasted_iota(jnp.int32, p.shape, 1)
    i1 = jnp.argmax(p, axis=-1, keepdims=True)
    m1 = jnp.max(p, axis=-1, keepdims=True)
    pm = jnp.where(col == i1, -jnp.inf, p)
    i2 = jnp.argmax(pm, axis=-1, keepdims=True)
    m2 = jnp.max(pm, axis=-1, keepdims=True)
    tot = m1 + m2
    comb_ref[...] = jnp.where(col == i1, m1 / tot,
                              jnp.where(col == i2, m2 / tot, 0.0))


def _moe_body(hn_ref, wg_ref, wu_ref, wd_ref, comb_ref, h_ref, o_ref):
    e = pl.program_id(1)
    hb = pl.program_id(2)

    @pl.when((e == 0) & (hb == 0))
    def _():
        o_ref[...] = h_ref[...]

    xb = hn_ref[...]
    dn = (((1,), (1,)), ((), ()))
    g = jax.lax.dot_general(xb, wg_ref[0], dn, preferred_element_type=jnp.float32)
    u = jax.lax.dot_general(xb, wu_ref[0], dn, preferred_element_type=jnp.float32)
    act = _silu(g) * u
    eo = jax.lax.dot_general(act, wd_ref[0], dn, preferred_element_type=jnp.float32)
    comb = comb_ref[...]
    col = jax.lax.broadcasted_iota(jnp.int32, comb.shape, 1)
    w_col = jnp.sum(jnp.where(col == e, comb, 0.0), axis=1, keepdims=True)
    o_ref[...] += eo * w_col


def kernel(x, ln1_w, ln1_b, ln2_w, ln2_b, in_proj_w, conv_w, conv_b, x_proj_w,
           dt_proj_w, dt_proj_b, A_log, Dp, out_proj_w, router_w, w_gate, w_up, w_down):
    b, t, d = x.shape
    di = conv_w.shape[0]
    r = dt_proj_w.shape[1]
    s = A_log.shape[1]
    e = router_w.shape[0]
    h_dim = w_gate.shape[1]
    n = b * t
    f32 = jnp.float32
    xf = x.reshape(n, d)

    ln = 256
    cb = 768
    n_nb = n // ln

    xz = pl.pallas_call(
        _inproj_body,
        grid=(n_nb, (2 * di) // cb),
        in_specs=[
            pl.BlockSpec((ln, d), lambda i, j: (i, 0)),
            pl.BlockSpec((d, cb), lambda i, j: (0, j)),
            pl.BlockSpec((1, d), lambda i, j: (0, 0)),
            pl.BlockSpec((1, d), lambda i, j: (0, 0)),
        ],
        out_specs=pl.BlockSpec((ln, cb), lambda i, j: (i, j)),
        out_shape=jax.ShapeDtypeStruct((n, 2 * di), f32),
    )(xf, in_proj_w.T, ln1_w.reshape(1, d), ln1_b.reshape(1, d))

    xs, dt, bp, cp = pl.pallas_call(
        _conv_body,
        grid=(n_nb,),
        in_specs=[
            pl.BlockSpec((ln, di), lambda i: (i, 0)),
            pl.BlockSpec((conv_w.shape[1], di), lambda i: (0, 0)),
            pl.BlockSpec((1, di), lambda i: (0, 0)),
            pl.BlockSpec((di, r + 2 * s), lambda i: (0, 0)),
            pl.BlockSpec((r, di), lambda i: (0, 0)),
            pl.BlockSpec((1, di), lambda i: (0, 0)),
        ],
        out_specs=[
            pl.BlockSpec((ln, di), lambda i: (i, 0)),
            pl.BlockSpec((ln, di), lambda i: (i, 0)),
            pl.BlockSpec((ln, s), lambda i: (i, 0)),
            pl.BlockSpec((ln, s), lambda i: (i, 0)),
        ],
        out_shape=[
            jax.ShapeDtypeStruct((n, di), f32),
            jax.ShapeDtypeStruct((n, di), f32),
            jax.ShapeDtypeStruct((n, s), f32),
            jax.ShapeDtypeStruct((n, s), f32),
        ],
        scratch_shapes=[pltpu.VMEM((8, di), f32)],
    )(xz, conv_w.T, conv_b.reshape(1, di), x_proj_w.T, dt_proj_w.T,
      dt_proj_b.reshape(1, di))

    lc = 256
    ys = pl.pallas_call(
        _scan_body,
        grid=(n // lc,),
        in_specs=[
            pl.BlockSpec((lc, di), lambda i: (i, 0)),
            pl.BlockSpec((lc, di), lambda i: (i, 0)),
            pl.BlockSpec((lc, s), lambda i: (i, 0)),
            pl.BlockSpec((lc, s), lambda i: (i, 0)),
            pl.BlockSpec((s, di), lambda i: (0, 0)),
            pl.BlockSpec((1, di), lambda i: (0, 0)),
        ],
        out_specs=pl.BlockSpec((lc, di), lambda i: (i, 0)),
        out_shape=jax.ShapeDtypeStruct((n, di), f32),
        scratch_shapes=[pltpu.VMEM((s, di), f32)],
    )(xs, dt, bp, cp, A_log.T, Dp.reshape(1, di))

    h, hn, comb = pl.pallas_call(
        _post_body,
        grid=(n_nb,),
        in_specs=[
            pl.BlockSpec((ln, di), lambda i: (i, 0)),
            pl.BlockSpec((ln, di), lambda i: (i, 1)),
            pl.BlockSpec((di, d), lambda i: (0, 0)),
            pl.BlockSpec((ln, d), lambda i: (i, 0)),
            pl.BlockSpec((1, d), lambda i: (0, 0)),
            pl.BlockSpec((1, d), lambda i: (0, 0)),
            pl.BlockSpec((d, e), lambda i: (0, 0)),
        ],
        out_specs=[
            pl.BlockSpec((ln, d), lambda i: (i, 0)),
            pl.BlockSpec((ln, d), lambda i: (i, 0)),
            pl.BlockSpec((ln, e), lambda i: (i, 0)),
        ],
        out_shape=[
            jax.ShapeDtypeStruct((n, d), f32),
            jax.ShapeDtypeStruct((n, d), f32),
            jax.ShapeDtypeStruct((n, e), f32),
        ],
    )(ys, xz, out_proj_w.T, xf, ln2_w.reshape(1, d), ln2_b.reshape(1, d),
      router_w.T)

    bn = 512
    bh = 512
    out = pl.pallas_call(
        _moe_body,
        grid=(n // bn, e, h_dim // bh),
        in_specs=[
            pl.BlockSpec((bn, d), lambda i, j, k: (i, 0)),
            pl.BlockSpec((1, bh, d), lambda i, j, k: (j, k, 0)),
            pl.BlockSpec((1, bh, d), lambda i, j, k: (j, k, 0)),
            pl.BlockSpec((1, d, bh), lambda i, j, k: (j, 0, k)),
            pl.BlockSpec((bn, e), lambda i, j, k: (i, 0)),
            pl.BlockSpec((bn, d), lambda i, j, k: (i, 0)),
        ],
        out_specs=pl.BlockSpec((bn, d), lambda i, j, k: (i, 0)),
        out_shape=jax.ShapeDtypeStruct((n, d), f32),
    )(hn, w_gate, w_up, w_down, comb, h)

    return out.reshape(b, t, d)


# speedup vs baseline: 14.0500x; 1.5752x over previous
import jax
import jax.numpy as jnp
from jax.experimental import pallas as pl
from jax.experimental.pallas import tpu as pltpu


def _silu(v):
    return v * jax.nn.sigmoid(v)


def _ln(v, w, b):
    m = v.mean(-1, keepdims=True)
    var = ((v - m) ** 2).mean(-1, keepdims=True)
    return (v - m) * jax.lax.rsqrt(var + 1e-5) * w + b


def _inproj_body(x_ref, w_ref, lnw_ref, lnb_ref, o_ref):
    xn = _ln(x_ref[...], lnw_ref[...], lnb_ref[...])
    o_ref[...] = jnp.dot(xn, w_ref[...], preferred_element_type=jnp.float32)


def _conv_body(xin_ref, convw_ref, convb_ref, xpw_ref, dtw_ref, dtb_ref,
               xs_ref, dt_ref, bp_ref, cp_ref, carry_ref):
    nb = pl.program_id(0)

    @pl.when(nb == 0)
    def _():
        carry_ref[...] = jnp.zeros_like(carry_ref)

    xin = xin_ref[...]
    ln = xin.shape[0]
    dc = convw_ref.shape[0]
    ext = jnp.concatenate([carry_ref[...], xin], axis=0)
    acc = jnp.broadcast_to(convb_ref[...], xin.shape)
    for k in range(dc):
        acc = acc + convw_ref[k, :][None, :] * ext[8 + k - (dc - 1): 8 + k - (dc - 1) + ln, :]
    xs = _silu(acc)
    xs_ref[...] = xs
    carry_ref[...] = xin[ln - 8: ln, :]
    xp = jnp.dot(xs, xpw_ref[...], preferred_element_type=jnp.float32)
    r = dtw_ref.shape[0]
    s = bp_ref.shape[1]
    bp_ref[...] = xp[:, r: r + s]
    cp_ref[...] = xp[:, r + s: r + 2 * s]
    dt_ref[...] = jax.nn.softplus(
        jnp.dot(xp[:, :r], dtw_ref[...], preferred_element_type=jnp.float32)
        + dtb_ref[...])


def _scan_body(xs_ref, dt_ref, bp_ref, cp_ref, alogt_ref, dp_ref, y_ref, h_ref):
    g = pl.program_id(0)

    @pl.when(g == 0)
    def _():
        h_ref[...] = jnp.zeros_like(h_ref)

    a_neg = -jnp.exp(alogt_ref[...])
    dp = dp_ref[...]
    lc = xs_ref.shape[0]
    u = 32
    outer_dn = (((0,), (0,)), ((), ()))
    contr_dn = (((1,), (0,)), ((), ()))

    def body(k, h):
        j0 = pl.multiple_of(k * u, u)
        dt_c = dt_ref[pl.ds(j0, u), :]
        xs_c = xs_ref[pl.ds(j0, u), :]
        bp_c = bp_ref[pl.ds(j0, u), :]
        cp_c = cp_ref[pl.ds(j0, u), :]
        dtx_c = dt_c * xs_c
        rows = []
        for j in range(u):
            dt_t = dt_c[j:j + 1, :]
            da = jnp.exp(dt_t * a_neg)
            dbx = jax.lax.dot_general(bp_c[j:j + 1, :], dtx_c[j:j + 1, :],
                                      outer_dn, preferred_element_type=jnp.float32)
            h = da * h + dbx
            rows.append(jax.lax.dot_general(cp_c[j:j + 1, :], h, contr_dn,
                                            preferred_element_type=jnp.float32))
        y_ref[pl.ds(j0, u), :] = jnp.concatenate(rows, axis=0) + dp * xs_c
        return h

    h_ref[...] = jax.lax.fori_loop(0, lc // u, body, h_ref[...])


def _post_body(ys_ref, z_ref, opw_ref, x_ref, ln2w_ref, ln2b_ref, rw_ref,
               h_ref, hn_ref, comb_ref):
    y = ys_ref[...] * _silu(z_ref[...])
    h = x_ref[...] + jnp.dot(y, opw_ref[...], preferred_element_type=jnp.float32)
    h_ref[...] = h
    hn = _ln(h, ln2w_ref[...], ln2b_ref[...])
    hn_ref[...] = hn
    logits = jnp.dot(hn, rw_ref[...], preferred_element_type=jnp.float32)
    p = jax.nn.softmax(logits, axis=-1)
    col = jax.lax.broadcasted_iota(jnp.int32, p.shape, 1)
    i1 = jnp.argmax(p, axis=-1, keepdims=True)
    m1 = jnp.max(p, axis=-1, keepdims=True)
    pm = jnp.where(col == i1, -jnp.inf, p)
    i2 = jnp.argmax(pm, axis=-1, keepdims=True)
    m2 = jnp.max(pm, axis=-1, keepdims=True)
    tot = m1 + m2
    comb_ref[...] = jnp.where(col == i1, m1 / tot,
                              jnp.where(col == i2, m2 / tot, 0.0))


def _moe_body(hn_ref, wg_ref, wu_ref, wd_ref, comb_ref, h_ref, o_ref):
    e = pl.program_id(1)
    hb = pl.program_id(2)

    @pl.when((e == 0) & (hb == 0))
    def _():
        o_ref[...] = h_ref[...]

    xb = hn_ref[...]
    dn = (((1,), (1,)), ((), ()))
    g = jax.lax.dot_general(xb, wg_ref[0], dn, preferred_element_type=jnp.float32)
    u = jax.lax.dot_general(xb, wu_ref[0], dn, preferred_element_type=jnp.float32)
    act = _silu(g) * u
    eo = jax.lax.dot_general(act, wd_ref[0], dn, preferred_element_type=jnp.float32)
    comb = comb_ref[...]
    col = jax.lax.broadcasted_iota(jnp.int32, comb.shape, 1)
    w_col = jnp.sum(jnp.where(col == e, comb, 0.0), axis=1, keepdims=True)
    o_ref[...] += eo * w_col


def kernel(x, ln1_w, ln1_b, ln2_w, ln2_b, in_proj_w, conv_w, conv_b, x_proj_w,
           dt_proj_w, dt_proj_b, A_log, Dp, out_proj_w, router_w, w_gate, w_up, w_down):
    b, t, d = x.shape
    di = conv_w.shape[0]
    r = dt_proj_w.shape[1]
    s = A_log.shape[1]
    e = router_w.shape[0]
    h_dim = w_gate.shape[1]
    n = b * t
    f32 = jnp.float32
    xf = x.reshape(n, d)

    ln = 256
    cb = 768
    n_nb = n // ln

    xz = pl.pallas_call(
        _inproj_body,
        grid=(n_nb, (2 * di) // cb),
        in_specs=[
            pl.BlockSpec((ln, d), lambda i, j: (i, 0)),
            pl.BlockSpec((d, cb), lambda i, j: (0, j)),
            pl.BlockSpec((1, d), lambda i, j: (0, 0)),
            pl.BlockSpec((1, d), lambda i, j: (0, 0)),
        ],
        out_specs=pl.BlockSpec((ln, cb), lambda i, j: (i, j)),
        out_shape=jax.ShapeDtypeStruct((n, 2 * di), f32),
    )(xf, in_proj_w.T, ln1_w.reshape(1, d), ln1_b.reshape(1, d))

    xs, dt, bp, cp = pl.pallas_call(
        _conv_body,
        grid=(n_nb,),
        in_specs=[
            pl.BlockSpec((ln, di), lambda i: (i, 0)),
            pl.BlockSpec((conv_w.shape[1], di), lambda i: (0, 0)),
            pl.BlockSpec((1, di), lambda i: (0, 0)),
            pl.BlockSpec((di, r + 2 * s), lambda i: (0, 0)),
            pl.BlockSpec((r, di), lambda i: (0, 0)),
            pl.BlockSpec((1, di), lambda i: (0, 0)),
        ],
        out_specs=[
            pl.BlockSpec((ln, di), lambda i: (i, 0)),
            pl.BlockSpec((ln, di), lambda i: (i, 0)),
            pl.BlockSpec((ln, s), lambda i: (i, 0)),
            pl.BlockSpec((ln, s), lambda i: (i, 0)),
        ],
        out_shape=[
            jax.ShapeDtypeStruct((n, di), f32),
            jax.ShapeDtypeStruct((n, di), f32),
            jax.ShapeDtypeStruct((n, s), f32),
            jax.ShapeDtypeStruct((n, s), f32),
        ],
        scratch_shapes=[pltpu.VMEM((8, di), f32)],
    )(xz, conv_w.T, conv_b.reshape(1, di), x_proj_w.T, dt_proj_w.T,
      dt_proj_b.reshape(1, di))

    lc = 256
    ys = pl.pallas_call(
        _scan_body,
        grid=(n // lc,),
        in_specs=[
            pl.BlockSpec((lc, di), lambda i: (i, 0)),
            pl.BlockSpec((lc, di), lambda i: (i, 0)),
            pl.BlockSpec((lc, s), lambda i: (i, 0)),
            pl.BlockSpec((lc, s), lambda i: (i, 0)),
            pl.BlockSpec((s, di), lambda i: (0, 0)),
            pl.BlockSpec((1, di), lambda i: (0, 0)),
        ],
        out_specs=pl.BlockSpec((lc, di), lambda i: (i, 0)),
        out_shape=jax.ShapeDtypeStruct((n, di), f32),
        scratch_shapes=[pltpu.VMEM((s, di), f32)],
    )(xs, dt, bp, cp, A_log.T, Dp.reshape(1, di))

    h, hn, comb = pl.pallas_call(
        _post_body,
        grid=(n_nb,),
        in_specs=[
            pl.BlockSpec((ln, di), lambda i: (i, 0)),
            pl.BlockSpec((ln, di), lambda i: (i, 1)),
            pl.BlockSpec((di, d), lambda i: (0, 0)),
            pl.BlockSpec((ln, d), lambda i: (i, 0)),
            pl.BlockSpec((1, d), lambda i: (0, 0)),
            pl.BlockSpec((1, d), lambda i: (0, 0)),
            pl.BlockSpec((d, e), lambda i: (0, 0)),
        ],
        out_specs=[
            pl.BlockSpec((ln, d), lambda i: (i, 0)),
            pl.BlockSpec((ln, d), lambda i: (i, 0)),
            pl.BlockSpec((ln, e), lambda i: (i, 0)),
        ],
        out_shape=[
            jax.ShapeDtypeStruct((n, d), f32),
            jax.ShapeDtypeStruct((n, d), f32),
            jax.ShapeDtypeStruct((n, e), f32),
        ],
    )(ys, xz, out_proj_w.T, xf, ln2_w.reshape(1, d), ln2_b.reshape(1, d),
      router_w.T)

    bn = 512
    bh = 512
    out = pl.pallas_call(
        _moe_body,
        grid=(n // bn, e, h_dim // bh),
        in_specs=[
            pl.BlockSpec((bn, d), lambda i, j, k: (i, 0)),
            pl.BlockSpec((1, bh, d), lambda i, j, k: (j, k, 0)),
            pl.BlockSpec((1, bh, d), lambda i, j, k: (j, k, 0)),
            pl.BlockSpec((1, d, bh), lambda i, j, k: (j, 0, k)),
            pl.BlockSpec((bn, e), lambda i, j, k: (i, 0)),
            pl.BlockSpec((bn, d), lambda i, j, k: (i, 0)),
        ],
        out_specs=pl.BlockSpec((bn, d), lambda i, j, k: (i, 0)),
        out_shape=jax.ShapeDtypeStruct((n, d), f32),
    )(hn, w_gate, w_up, w_down, comb, h)

    return out.reshape(b, t, d)


# speedup vs baseline: 21.5444x; 1.5334x over previous
import jax
import jax.numpy as jnp
from jax.experimental import pallas as pl
from jax.experimental.pallas import tpu as pltpu


def _silu(v):
    return v * jax.nn.sigmoid(v)


def _ln(v, w, b):
    m = v.mean(-1, keepdims=True)
    var = ((v - m) ** 2).mean(-1, keepdims=True)
    return (v - m) * jax.lax.rsqrt(var + 1e-5) * w + b


def _inproj_body(x_ref, w_ref, lnw_ref, lnb_ref, o_ref):
    xn = _ln(x_ref[...], lnw_ref[...], lnb_ref[...])
    o_ref[...] = jnp.dot(xn, w_ref[...], preferred_element_type=jnp.float32)


def _conv_body(xin_ref, convw_ref, convb_ref, xpw_ref, dtw_ref, dtb_ref,
               xs_ref, dt_ref, bp_ref, cp_ref, carry_ref):
    nb = pl.program_id(0)

    @pl.when(nb == 0)
    def _():
        carry_ref[...] = jnp.zeros_like(carry_ref)

    xin = xin_ref[...]
    ln = xin.shape[0]
    dc = convw_ref.shape[0]
    ext = jnp.concatenate([carry_ref[...], xin], axis=0)
    acc = jnp.broadcast_to(convb_ref[...], xin.shape)
    for k in range(dc):
        acc = acc + convw_ref[k, :][None, :] * ext[8 + k - (dc - 1): 8 + k - (dc - 1) + ln, :]
    xs = _silu(acc)
    xs_ref[...] = xs
    carry_ref[...] = xin[ln - 8: ln, :]
    xp = jnp.dot(xs, xpw_ref[...], preferred_element_type=jnp.float32)
    r = dtw_ref.shape[0]
    s = bp_ref.shape[1]
    bp_ref[...] = xp[:, r: r + s]
    cp_ref[...] = xp[:, r + s: r + 2 * s]
    dt_ref[...] = jax.nn.softplus(
        jnp.dot(xp[:, :r], dtw_ref[...], preferred_element_type=jnp.float32)
        + dtb_ref[...])


def _scan_body(xs_ref, dt_ref, bp_ref, cp_ref, alogt_ref, dp_ref, y_ref,
               h_ref, da_ref, dbx_ref):
    g = pl.program_id(0)

    @pl.when(g == 0)
    def _():
        h_ref[...] = jnp.zeros_like(h_ref)

    a_neg = -jnp.exp(alogt_ref[...])
    dp = dp_ref[...]
    lc = xs_ref.shape[0]
    u = 32
    outer_dn = (((0,), (0,)), ((), ()))
    contr_dn = (((1,), (0,)), ((), ()))

    def body(k, h):
        j0 = pl.multiple_of(k * u, u)
        dt_c = dt_ref[pl.ds(j0, u), :]
        xs_c = xs_ref[pl.ds(j0, u), :]
        bp_c = bp_ref[pl.ds(j0, u), :]
        cp_c = cp_ref[pl.ds(j0, u), :]
        dtx_c = dt_c * xs_c
        for j in range(u):
            da_ref[j] = jnp.exp(dt_c[j:j + 1, :] * a_neg)
            dbx_ref[j] = jax.lax.dot_general(
                bp_c[j:j + 1, :], dtx_c[j:j + 1, :], outer_dn,
                preferred_element_type=jnp.float32)
        rows = []
        for j in range(u):
            h = da_ref[j] * h + dbx_ref[j]
            rows.append(jax.lax.dot_general(cp_c[j:j + 1, :], h, contr_dn,
                                            preferred_element_type=jnp.float32))
        y_ref[pl.ds(j0, u), :] = jnp.concatenate(rows, axis=0) + dp * xs_c
        return h

    h_ref[...] = jax.lax.fori_loop(0, lc // u, body, h_ref[...])


def _post_body(ys_ref, z_ref, opw_ref, x_ref, ln2w_ref, ln2b_ref, rw_ref,
               h_ref, hn_ref, comb_ref):
    y = ys_ref[...] * _silu(z_ref[...])
    h = x_ref[...] + jnp.dot(y, opw_ref[...], preferred_element_type=jnp.float32)
    h_ref[...] = h
    hn = _ln(h, ln2w_ref[...], ln2b_ref[...])
    hn_ref[...] = hn
    logits = jnp.dot(hn, rw_ref[...], preferred_element_type=jnp.float32)
    p = jax.nn.softmax(logits, axis=-1)
    col = jax.lax.broadcasted_iota(jnp.int32, p.shape, 1)
    i1 = jnp.argmax(p, axis=-1, keepdims=True)
    m1 = jnp.max(p, axis=-1, keepdims=True)
    pm = jnp.where(col == i1, -jnp.inf, p)
    i2 = jnp.argmax(pm, axis=-1, keepdims=True)
    m2 = jnp.max(pm, axis=-1, keepdims=True)
    tot = m1 + m2
    comb_ref[...] = jnp.where(col == i1, m1 / tot,
                              jnp.where(col == i2, m2 / tot, 0.0))


def _moe_body(hn_ref, wg_ref, wu_ref, wd_ref, comb_ref, h_ref, o_ref):
    e = pl.program_id(1)
    hb = pl.program_id(2)

    @pl.when((e == 0) & (hb == 0))
    def _():
        o_ref[...] = h_ref[...]

    xb = hn_ref[...]
    dn = (((1,), (1,)), ((), ()))
    g = jax.lax.dot_general(xb, wg_ref[0], dn, preferred_element_type=jnp.float32)
    u = jax.lax.dot_general(xb, wu_ref[0], dn, preferred_element_type=jnp.float32)
    act = _silu(g) * u
    eo = jax.lax.dot_general(act, wd_ref[0], dn, preferred_element_type=jnp.float32)
    comb = comb_ref[...]
    col = jax.lax.broadcasted_iota(jnp.int32, comb.shape, 1)
    w_col = jnp.sum(jnp.where(col == e, comb, 0.0), axis=1, keepdims=True)
    o_ref[...] += eo * w_col


def kernel(x, ln1_w, ln1_b, ln2_w, ln2_b, in_proj_w, conv_w, conv_b, x_proj_w,
           dt_proj_w, dt_proj_b, A_log, Dp, out_proj_w, router_w, w_gate, w_up, w_down):
    b, t, d = x.shape
    di = conv_w.shape[0]
    r = dt_proj_w.shape[1]
    s = A_log.shape[1]
    e = router_w.shape[0]
    h_dim = w_gate.shape[1]
    n = b * t
    f32 = jnp.float32
    xf = x.reshape(n, d)

    ln = 256
    cb = 768
    n_nb = n // ln

    xz = pl.pallas_call(
        _inproj_body,
        grid=(n_nb, (2 * di) // cb),
        in_specs=[
            pl.BlockSpec((ln, d), lambda i, j: (i, 0)),
            pl.BlockSpec((d, cb), lambda i, j: (0, j)),
            pl.BlockSpec((1, d), lambda i, j: (0, 0)),
            pl.BlockSpec((1, d), lambda i, j: (0, 0)),
        ],
        out_specs=pl.BlockSpec((ln, cb), lambda i, j: (i, j)),
        out_shape=jax.ShapeDtypeStruct((n, 2 * di), f32),
    )(xf, in_proj_w.T, ln1_w.reshape(1, d), ln1_b.reshape(1, d))

    xs, dt, bp, cp = pl.pallas_call(
        _conv_body,
        grid=(n_nb,),
        in_specs=[
            pl.BlockSpec((ln, di), lambda i: (i, 0)),
            pl.BlockSpec((conv_w.shape[1], di), lambda i: (0, 0)),
            pl.BlockSpec((1, di), lambda i: (0, 0)),
            pl.BlockSpec((di, r + 2 * s), lambda i: (0, 0)),
            pl.BlockSpec((r, di), lambda i: (0, 0)),
            pl.BlockSpec((1, di), lambda i: (0, 0)),
        ],
        out_specs=[
            pl.BlockSpec((ln, di), lambda i: (i, 0)),
            pl.BlockSpec((ln, di), lambda i: (i, 0)),
            pl.BlockSpec((ln, s), lambda i: (i, 0)),
            pl.BlockSpec((ln, s), lambda i: (i, 0)),
        ],
        out_shape=[
            jax.ShapeDtypeStruct((n, di), f32),
            jax.ShapeDtypeStruct((n, di), f32),
            jax.ShapeDtypeStruct((n, s), f32),
            jax.ShapeDtypeStruct((n, s), f32),
        ],
        scratch_shapes=[pltpu.VMEM((8, di), f32)],
    )(xz, conv_w.T, conv_b.reshape(1, di), x_proj_w.T, dt_proj_w.T,
      dt_proj_b.reshape(1, di))

    lc = 256
    ys = pl.pallas_call(
        _scan_body,
        grid=(n // lc,),
        in_specs=[
            pl.BlockSpec((lc, di), lambda i: (i, 0)),
            pl.BlockSpec((lc, di), lambda i: (i, 0)),
            pl.BlockSpec((lc, s), lambda i: (i, 0)),
            pl.BlockSpec((lc, s), lambda i: (i, 0)),
            pl.BlockSpec((s, di), lambda i: (0, 0)),
            pl.BlockSpec((1, di), lambda i: (0, 0)),
        ],
        out_specs=pl.BlockSpec((lc, di), lambda i: (i, 0)),
        out_shape=jax.ShapeDtypeStruct((n, di), f32),
        scratch_shapes=[pltpu.VMEM((s, di), f32),
                        pltpu.VMEM((32, s, di), f32),
                        pltpu.VMEM((32, s, di), f32)],
    )(xs, dt, bp, cp, A_log.T, Dp.reshape(1, di))

    h, hn, comb = pl.pallas_call(
        _post_body,
        grid=(n_nb,),
        in_specs=[
            pl.BlockSpec((ln, di), lambda i: (i, 0)),
            pl.BlockSpec((ln, di), lambda i: (i, 1)),
            pl.BlockSpec((di, d), lambda i: (0, 0)),
            pl.BlockSpec((ln, d), lambda i: (i, 0)),
            pl.BlockSpec((1, d), lambda i: (0, 0)),
            pl.BlockSpec((1, d), lambda i: (0, 0)),
            pl.BlockSpec((d, e), lambda i: (0, 0)),
        ],
        out_specs=[
            pl.BlockSpec((ln, d), lambda i: (i, 0)),
            pl.BlockSpec((ln, d), lambda i: (i, 0)),
            pl.BlockSpec((ln, e), lambda i: (i, 0)),
        ],
        out_shape=[
            jax.ShapeDtypeStruct((n, d), f32),
            jax.ShapeDtypeStruct((n, d), f32),
            jax.ShapeDtypeStruct((n, e), f32),
        ],
    )(ys, xz, out_proj_w.T, xf, ln2_w.reshape(1, d), ln2_b.reshape(1, d),
      router_w.T)

    bn = 512
    bh = 512
    out = pl.pallas_call(
        _moe_body,
        grid=(n // bn, e, h_dim // bh),
        in_specs=[
            pl.BlockSpec((bn, d), lambda i, j, k: (i, 0)),
            pl.BlockSpec((1, bh, d), lambda i, j, k: (j, k, 0)),
            pl.BlockSpec((1, bh, d), lambda i, j, k: (j, k, 0)),
            pl.BlockSpec((1, d, bh), lambda i, j, k: (j, 0, k)),
            pl.BlockSpec((bn, e), lambda i, j, k: (i, 0)),
            pl.BlockSpec((bn, d), lambda i, j, k: (i, 0)),
        ],
        out_specs=pl.BlockSpec((bn, d), lambda i, j, k: (i, 0)),
        out_shape=jax.ShapeDtypeStruct((n, d), f32),
    )(hn, w_gate, w_up, w_down, comb, h)

    return out.reshape(b, t, d)
